# baseline (speedup 1.0000x reference)
import jax
import jax.numpy as jnp
from jax import lax
from jax.experimental import pallas as pl
from jax.experimental.pallas import tpu as pltpu
from jax.experimental.pallas import tpu_sc as plsc

B, T_HALF, D = 4, 1024, 1024
T = 2 * T_HALF
NW = 32
ROWS_PER_W = T // NW
SUB = 16
NSUB = ROWS_PER_W // SUB
LANES = 16


def _sc_body(seq_a, seq_b, emb, out, tab_v, buf0, buf1,
             sem_t, sem_l0, sem_l1, sem_o0, sem_o1):
    cid = lax.axis_index("c")
    sid = lax.axis_index("s")
    wid = sid * 2 + cid
    half = wid // 16
    r0 = (wid % 16) * ROWS_PER_W
    g0 = wid * ROWS_PER_W
    bufs = (buf0, buf1)
    sem_l = (sem_l0, sem_l1)
    sem_o = (sem_o0, sem_o1)

    def load_seq(b, s, k):
        @pl.when(half == 0)
        def _():
            pltpu.async_copy(seq_a.at[b, pl.ds(r0 + SUB * s, SUB), :],
                             bufs[k], sem_l[k])

        @pl.when(half == 1)
        def _():
            pltpu.async_copy(seq_b.at[b, pl.ds(r0 + SUB * s, SUB), :],
                             bufs[k], sem_l[k])

    def wait_seq(b, s, k):
        pltpu.make_async_copy(seq_a.at[b, pl.ds(r0 + SUB * s, SUB), :],
                              bufs[k], sem_l[k]).wait()

    def store_out(b, s, k):
        pltpu.async_copy(bufs[k], out.at[b, pl.ds(g0 + SUB * s, SUB), :],
                         sem_o[k])

    def wait_out(b, s, k):
        pltpu.make_async_copy(bufs[k], out.at[b, pl.ds(g0 + SUB * s, SUB), :],
                              sem_o[k]).wait()

    def add_table(k):
        buf = bufs[k]

        def row(r, carry):
            for j in range(D // LANES):
                sl = pl.ds(j * LANES, LANES)
                buf[r, sl] = buf[r, sl] + tab_v[r, sl]
            return carry

        lax.fori_loop(0, SUB, row, 0)

    last_store = [None, None]

    for s in range(NSUB):
        pltpu.async_copy(emb.at[pl.ds(g0 + SUB * s, SUB), :], tab_v, sem_t)
        if last_store[0] is not None:
            wait_out(*last_store[0])
            last_store[0] = None
        load_seq(0, s, 0)
        pltpu.make_async_copy(emb.at[pl.ds(g0 + SUB * s, SUB), :],
                              tab_v, sem_t).wait()
        for b in range(B):
            k = b % 2
            if b + 1 < B:
                kn = (b + 1) % 2
                if last_store[kn] is not None:
                    wait_out(*last_store[kn])
                    last_store[kn] = None
                load_seq(b + 1, s, kn)
            wait_seq(b, s, k)
            add_table(k)
            store_out(b, s, k)
            last_store[k] = (b, s, k)

    for k in range(2):
        if last_store[k] is not None:
            wait_out(*last_store[k])


def kernel(seq_a, seq_b, emb_table):
    mesh = plsc.VectorSubcoreMesh(core_axis_name="c", subcore_axis_name="s")
    f = pl.kernel(
        _sc_body,
        out_type=jax.ShapeDtypeStruct((B, T, D), jnp.float32),
        mesh=mesh,
        scratch_types=[
            pltpu.VMEM((SUB, D), jnp.float32),
            pltpu.VMEM((SUB, D), jnp.float32),
            pltpu.VMEM((SUB, D), jnp.float32),
            pltpu.SemaphoreType.DMA,
            pltpu.SemaphoreType.DMA,
            pltpu.SemaphoreType.DMA,
            pltpu.SemaphoreType.DMA,
            pltpu.SemaphoreType.DMA,
        ],
    )
    return f(seq_a, seq_b, emb_table)


# speedup vs baseline: 1.1781x; 1.1781x over previous
import jax
import jax.numpy as jnp
from jax import lax
from jax.experimental import pallas as pl
from jax.experimental.pallas import tpu as pltpu
from jax.experimental.pallas import tpu_sc as plsc

B, T_HALF, D = 4, 1024, 1024
T = 2 * T_HALF
NW = 32
ROWS_PER_W = T // NW
SUB = 16
NSUB = ROWS_PER_W // SUB
NBUF = 5
AHEAD = 3
LANES = 16

_CHUNKS = [(s, b) for s in range(NSUB) for b in range(B)]


def _sc_body(seq_a, seq_b, emb, out,
             tab0, tab1, buf0, buf1, buf2, buf3, buf4,
             smt0, smt1,
             sml0, sml1, sml2, sml3, sml4,
             smo0, smo1, smo2, smo3, smo4):
    cid = lax.axis_index("c")
    sid = lax.axis_index("s")
    wid = sid * 2 + cid
    half = wid // 16
    r0 = (wid % 16) * ROWS_PER_W
    g0 = wid * ROWS_PER_W
    tabs = (tab0, tab1)
    bufs = (buf0, buf1, buf2, buf3, buf4)
    sem_t = (smt0, smt1)
    sem_l = (sml0, sml1, sml2, sml3, sml4)
    sem_o = (smo0, smo1, smo2, smo3, smo4)

    def load_tab(s):
        kt = s % 2
        pltpu.async_copy(emb.at[pl.ds(g0 + SUB * s, SUB), :],
                         tabs[kt], sem_t[kt])

    def wait_tab(s):
        kt = s % 2
        pltpu.make_async_copy(emb.at[pl.ds(g0 + SUB * s, SUB), :],
                              tabs[kt], sem_t[kt]).wait()

    def load_seq(i):
        s, b = _CHUNKS[i]
        k = i % NBUF

        @pl.when(half == 0)
        def _():
            pltpu.async_copy(seq_a.at[b, pl.ds(r0 + SUB * s, SUB), :],
                             bufs[k], sem_l[k])

        @pl.when(half == 1)
        def _():
            pltpu.async_copy(seq_b.at[b, pl.ds(r0 + SUB * s, SUB), :],
                             bufs[k], sem_l[k])

    def wait_seq(i):
        s, b = _CHUNKS[i]
        k = i % NBUF
        pltpu.make_async_copy(seq_a.at[b, pl.ds(r0 + SUB * s, SUB), :],
                              bufs[k], sem_l[k]).wait()

    def store_out(i):
        s, b = _CHUNKS[i]
        k = i % NBUF
        pltpu.async_copy(bufs[k], out.at[b, pl.ds(g0 + SUB * s, SUB), :],
                         sem_o[k])

    def wait_out(i):
        s, b = _CHUNKS[i]
        k = i % NBUF
        pltpu.make_async_copy(bufs[k], out.at[b, pl.ds(g0 + SUB * s, SUB), :],
                              sem_o[k]).wait()

    def add_tab(i):
        s, _ = _CHUNKS[i]
        buf = bufs[i % NBUF]
        tab = tabs[s % 2]

        def row(r, carry):
            for j in range(D // LANES):
                sl = pl.ds(j * LANES, LANES)
                buf[r, sl] = buf[r, sl] + tab[r, sl]
            return carry

        lax.fori_loop(0, SUB, row, 0)

    n = len(_CHUNKS)
    last_store = [None] * NBUF

    load_tab(0)
    for i in range(min(AHEAD, n)):
        load_seq(i)

    for i in range(n):
        s, b = _CHUNKS[i]
        if b == 0:
            if s + 1 < NSUB:
                load_tab(s + 1)
            wait_tab(s)
        j = i + AHEAD
        if j < n:
            kj = j % NBUF
            if last_store[kj] is not None:
                wait_out(last_store[kj])
                last_store[kj] = None
            load_seq(j)
        wait_seq(i)
        add_tab(i)
        store_out(i)
        last_store[i % NBUF] = i

    for k in range(NBUF):
        if last_store[k] is not None:
            wait_out(last_store[k])


def kernel(seq_a, seq_b, emb_table):
    mesh = plsc.VectorSubcoreMesh(core_axis_name="c", subcore_axis_name="s")
    f = pl.kernel(
        _sc_body,
        out_type=jax.ShapeDtypeStruct((B, T, D), jnp.float32),
        mesh=mesh,
        scratch_types=(
            [pltpu.VMEM((SUB, D), jnp.float32) for _ in range(2)]
            + [pltpu.VMEM((SUB, D), jnp.float32) for _ in range(NBUF)]
            + [pltpu.SemaphoreType.DMA for _ in range(2 + 2 * NBUF)]
        ),
    )
    return f(seq_a, seq_b, emb_table)


# speedup vs baseline: 1.2469x; 1.0584x over previous
import jax
import jax.numpy as jnp
from jax import lax
from jax.experimental import pallas as pl
from jax.experimental.pallas import tpu as pltpu
from jax.experimental.pallas import tpu_sc as plsc

B, T_HALF, D = 4, 1024, 1024
T = 2 * T_HALF
NW = 32
ROWS_PER_W = T // NW
SUB = 8
NG = ROWS_PER_W // SUB
LANES = 16
HALF_VREGS = 32


def _sc_body(seq_a, seq_b, emb, out,
             tab0, tab1,
             a00, a01, a02, a03, a10, a11, a12, a13,
             smt0, smt1,
             sl00, sl01, sl02, sl03, sl10, sl11, sl12, sl13,
             so00, so01, so02, so03, so10, so11, so12, so13):
    cid = lax.axis_index("c")
    sid = lax.axis_index("s")
    wid = sid * 2 + cid
    half = wid // 16
    r0 = (wid % 16) * ROWS_PER_W
    g0 = wid * ROWS_PER_W
    tabs = (tab0, tab1)
    bufs = ((a00, a01, a02, a03), (a10, a11, a12, a13))
    sem_t = (smt0, smt1)
    sem_l = ((sl00, sl01, sl02, sl03), (sl10, sl11, sl12, sl13))
    sem_o = ((so00, so01, so02, so03), (so10, so11, so12, so13))

    def load_group(g):
        p = g % 2
        pltpu.async_copy(emb.at[pl.ds(g0 + SUB * g, SUB), :],
                         tabs[p], sem_t[p])
        for b in range(B):
            @pl.when(half == 0)
            def _():
                pltpu.async_copy(seq_a.at[b, pl.ds(r0 + SUB * g, SUB), :],
                                 bufs[p][b], sem_l[p][b])

            @pl.when(half == 1)
            def _():
                pltpu.async_copy(seq_b.at[b, pl.ds(r0 + SUB * g, SUB), :],
                                 bufs[p][b], sem_l[p][b])

    def wait_group_loads(g):
        p = g % 2
        pltpu.make_async_copy(emb.at[pl.ds(g0 + SUB * g, SUB), :],
                              tabs[p], sem_t[p]).wait()
        for b in range(B):
            pltpu.make_async_copy(seq_a.at[b, pl.ds(r0 + SUB * g, SUB), :],
                                  bufs[p][b], sem_l[p][b]).wait()

    def store_group(g):
        p = g % 2
        for b in range(B):
            pltpu.async_copy(bufs[p][b],
                             out.at[b, pl.ds(g0 + SUB * g, SUB), :],
                             sem_o[p][b])

    def wait_group_stores(g):
        p = g % 2
        for b in range(B):
            pltpu.make_async_copy(bufs[p][b],
                                  out.at[b, pl.ds(g0 + SUB * g, SUB), :],
                                  sem_o[p][b]).wait()

    def compute_group(g):
        p = g % 2
        tab = tabs[p]
        gb = bufs[p]

        def row(r, carry):
            for h in range(D // (HALF_VREGS * LANES)):
                base = h * HALF_VREGS * LANES
                tvs = [tab[r, pl.ds(base + j * LANES, LANES)]
                       for j in range(HALF_VREGS)]
                for b in range(B):
                    buf = gb[b]
                    for j in range(HALF_VREGS):
                        sl = pl.ds(base + j * LANES, LANES)
                        buf[r, sl] = buf[r, sl] + tvs[j]
            return carry

        lax.fori_loop(0, SUB, row, 0)

    load_group(0)
    for g in range(NG):
        if g + 1 < NG:
            if g >= 1:
                wait_group_stores(g - 1)
            load_group(g + 1)
        wait_group_loads(g)
        compute_group(g)
        store_group(g)

    wait_group_stores(NG - 2)
    wait_group_stores(NG - 1)


def kernel(seq_a, seq_b, emb_table):
    mesh = plsc.VectorSubcoreMesh(core_axis_name="c", subcore_axis_name="s")
    f = pl.kernel(
        _sc_body,
        out_type=jax.ShapeDtypeStruct((B, T, D), jnp.float32),
        mesh=mesh,
        scratch_types=(
            [pltpu.VMEM((SUB, D), jnp.float32) for _ in range(2)]
            + [pltpu.VMEM((SUB, D), jnp.float32) for _ in range(2 * B)]
            + [pltpu.SemaphoreType.DMA for _ in range(2 + 4 * B)]
        ),
    )
    return f(seq_a, seq_b, emb_table)
